# Initial kernel scaffold; baseline (speedup 1.0000x reference)
import jax
import jax.numpy as jnp
from jax.experimental import pallas as pl


def kernel(logits):
    raise NotImplementedError("write your pallas kernel here")


# speedup vs baseline: 1.1687x; 1.1687x over previous
import functools

import jax
import jax.numpy as jnp
from jax import lax
from jax.experimental import pallas as pl
from jax.experimental.pallas import tpu as pltpu
from jax.experimental.pallas import tpu_sc as plsc

B = 128
SLEN = 8192
K_SEL = 128

NC = 2
NS = 16
L = 16
NW = NC * NS
ROWS_PER_W = B // NW
NVEC = SLEN // L

_SIGN = jnp.int32(-2147483648)


def _splat_u32(x):
    return jnp.full((L,), x, dtype=jnp.uint32)


@functools.partial(
    pl.kernel,
    out_type=jax.ShapeDtypeStruct((B, SLEN), jnp.float32),
    mesh=plsc.VectorSubcoreMesh(core_axis_name="c", subcore_axis_name="s"),
    compiler_params=pltpu.CompilerParams(needs_layout_passes=False),
    scratch_types=[
        pltpu.VMEM((ROWS_PER_W, SLEN), jnp.float32),
        pltpu.VMEM((ROWS_PER_W, SLEN), jnp.uint32),
        pltpu.VMEM((ROWS_PER_W, SLEN), jnp.float32),
    ],
)
def _topk_mask_sc(x_hbm, out_hbm, xf, xu, of):
    wid = lax.axis_index("s") * NC + lax.axis_index("c")
    base = wid * ROWS_PER_W

    pltpu.sync_copy(x_hbm.at[pl.ds(base, ROWS_PER_W)], xf)

    for r in range(ROWS_PER_W):
        def map_body(i, _):
            v = xf[r, pl.ds(i * L, L)]
            bi = lax.bitcast_convert_type(v, jnp.int32)
            s = lax.shift_right_arithmetic(bi, jnp.int32(31))
            u = lax.bitwise_xor(bi, lax.bitwise_or(s, _SIGN))
            xu[r, pl.ds(i * L, L)] = lax.bitcast_convert_type(u, jnp.uint32)
            return 0

        lax.fori_loop(0, NVEC, map_body, 0)

        one = jnp.ones((L,), jnp.int32)
        zero = jnp.zeros((L,), jnp.int32)

        def bit_body(i, lo):
            shift = jnp.full((L,), 31 - i, dtype=jnp.uint32)
            mid = lo + (_splat_u32(1) << shift)

            def cnt_body(j, cnt):
                u = xu[r, pl.ds(j * L, L)]
                m = u >= mid
                return cnt + jnp.where(m, one, zero)

            cnt = lax.fori_loop(0, NVEC, cnt_body, zero)
            c = jnp.sum(cnt)
            return jnp.where(c >= K_SEL, mid, lo)

        lo = lax.fori_loop(0, 32, bit_body, _splat_u32(0))

        lo_i = lax.bitcast_convert_type(lo, jnp.int32)
        was_pos = lo_i < 0
        bits = jnp.where(
            was_pos,
            lax.bitwise_xor(lo_i, _SIGN),
            lax.bitwise_not(lo_i),
        )
        tf = lax.bitcast_convert_type(bits, jnp.float32)

        def mask_body(i, _):
            v = xf[r, pl.ds(i * L, L)]
            of[r, pl.ds(i * L, L)] = jnp.where(
                v >= tf, jnp.float32(1.0), jnp.float32(0.0)
            )
            return 0

        lax.fori_loop(0, NVEC, mask_body, 0)

    pltpu.sync_copy(of, out_hbm.at[pl.ds(base, ROWS_PER_W)])


def kernel(logits):
    x = logits.reshape(B, SLEN)
    y = _topk_mask_sc(x)
    return y[..., None]


# speedup vs baseline: 3.1383x; 2.6854x over previous
import functools

import jax
import jax.numpy as jnp
from jax import lax
from jax.experimental import pallas as pl
from jax.experimental.pallas import tpu as pltpu
from jax.experimental.pallas import tpu_sc as plsc

B = 128
SLEN = 8192
K_SEL = 128

NC = 2
NS = 16
L = 16
NW = NC * NS
ROWS_PER_W = B // NW
NVEC = SLEN // L
CAND = SLEN + 40 * L

_SIGN = jnp.int32(-2147483648)


@functools.partial(
    pl.kernel,
    out_type=jax.ShapeDtypeStruct((B * SLEN,), jnp.float32),
    mesh=plsc.VectorSubcoreMesh(core_axis_name="c", subcore_axis_name="s"),
    compiler_params=pltpu.CompilerParams(needs_layout_passes=False),
    scratch_types=[
        pltpu.VMEM((ROWS_PER_W * SLEN,), jnp.float32),
        pltpu.VMEM((CAND,), jnp.uint32),
        pltpu.VMEM((CAND,), jnp.uint32),
    ],
)
def _topk_mask_sc(x_hbm, out_hbm, xf, ca, cb):
    wid = lax.axis_index("s") * NC + lax.axis_index("c")
    base = wid * ROWS_PER_W

    pltpu.sync_copy(x_hbm.at[pl.ds(base * SLEN, ROWS_PER_W * SLEN)], xf)

    one = jnp.ones((L,), jnp.int32)
    zero = jnp.zeros((L,), jnp.int32)
    zero_u = jnp.zeros((L,), jnp.uint32)

    def select_step(bit, state, src, dst):
        lo, cnt_hi, n = state
        shift = jnp.full((L,), bit, dtype=jnp.uint32)
        mid = lo + (jnp.full((L,), 1, jnp.uint32) << shift)
        nv = (n + (L - 1)) // L

        def cnt_body(j, cnt):
            u = src[pl.ds(j * L, L)]
            return cnt + jnp.where(u >= mid, one, zero)

        c = jnp.sum(lax.fori_loop(0, nv, cnt_body, zero))
        keep_hi = (cnt_hi + c) >= K_SEL

        def cmp_body(j, pos):
            u = src[pl.ds(j * L, L)]
            m = u >= mid
            sel = jnp.where(keep_hi, m, ~m)
            plsc.store_compressed(dst.at[pl.ds(pos, L)], u, mask=sel)
            return pos + jnp.sum(jnp.where(sel, one, zero))

        pos = lax.fori_loop(0, nv, cmp_body, jnp.int32(0))
        dst[pl.ds(pos, L)] = zero_u

        lo = jnp.where(keep_hi, mid, lo)
        cnt_hi = jnp.where(keep_hi, cnt_hi, cnt_hi + c)
        return lo, cnt_hi, pos

    def row_body(r, _):
        rb = r * SLEN

        def map_body(i, _):
            v = xf[pl.ds(rb + i * L, L)]
            bi = lax.bitcast_convert_type(v, jnp.int32)
            s = lax.shift_right_arithmetic(bi, jnp.int32(31))
            u = lax.bitwise_xor(bi, lax.bitwise_or(s, _SIGN))
            ca[pl.ds(i * L, L)] = lax.bitcast_convert_type(u, jnp.uint32)
            return 0

        lax.fori_loop(0, NVEC, map_body, 0)

        def bit_body(t, state):
            state = select_step(31 - 2 * t, state, ca, cb)
            return select_step(30 - 2 * t, state, cb, ca)

        init = (jnp.zeros((L,), jnp.uint32), jnp.int32(0), jnp.int32(SLEN))
        lo, _, _ = lax.fori_loop(0, 16, bit_body, init)

        lo_i = lax.bitcast_convert_type(lo, jnp.int32)
        was_pos = lo_i < 0
        bits = jnp.where(
            was_pos,
            lax.bitwise_xor(lo_i, _SIGN),
            lax.bitwise_not(lo_i),
        )
        tf = lax.bitcast_convert_type(bits, jnp.float32)

        def mask_body(i, _):
            v = xf[pl.ds(rb + i * L, L)]
            xf[pl.ds(rb + i * L, L)] = jnp.where(
                v >= tf, jnp.float32(1.0), jnp.float32(0.0)
            )
            return 0

        lax.fori_loop(0, NVEC, mask_body, 0)
        return 0

    lax.fori_loop(0, ROWS_PER_W, row_body, 0)

    pltpu.sync_copy(xf, out_hbm.at[pl.ds(base * SLEN, ROWS_PER_W * SLEN)])


def kernel(logits):
    x = logits.reshape(B * SLEN)
    y = _topk_mask_sc(x)
    return y.reshape(B, SLEN, 1)
